# baseline (speedup 1.0000x reference)
import functools

import jax
import jax.numpy as jnp
from jax import lax
from jax.experimental import pallas as pl
from jax.experimental.pallas import tpu as pltpu
from jax.experimental.pallas import tpu_sc as plsc

VOCAB = 1000000
SEQ = 200
DIM = 64
BATCH = 4096

NC = 2
NS = 16
NW = NC * NS
LANES = 16

ROWS = BATCH * SEQ
RPW = ROWS // NW
CHUNK = 128
NCHUNK = RPW // CHUNK
NBUF = 8
LEAD = 4


@functools.partial(
    pl.kernel,
    mesh=plsc.VectorSubcoreMesh(core_axis_name="c", subcore_axis_name="s"),
    out_type=jax.ShapeDtypeStruct((ROWS, DIM), jnp.float32),
    compiler_params=pltpu.CompilerParams(use_tc_tiling_on_sc=False),
    scratch_types=[
        pltpu.VMEM((NCHUNK, CHUNK), jnp.int32),
        pltpu.VMEM((2 * SEQ, DIM), jnp.float32),
        pltpu.VMEM((NBUF, CHUNK, DIM), jnp.float32),
        pltpu.SemaphoreType.DMA((NBUF,)),
        pltpu.SemaphoreType.DMA((NBUF,)),
    ],
)
def _emb_kernel(idx_hbm, pos_hbm, table_hbm, out_hbm,
                idx_v, pos_v, rows_v, sem_g, sem_s):
    wid = lax.axis_index("s") * NC + lax.axis_index("c")
    base_row = wid * RPW

    pltpu.sync_copy(idx_hbm.at[pl.ds(wid * NCHUNK, NCHUNK)], idx_v)
    pltpu.sync_copy(pos_hbm, pos_v.at[pl.ds(0, SEQ)])
    pltpu.sync_copy(pos_hbm, pos_v.at[pl.ds(SEQ, SEQ)])

    def gather_copy(c, slot):
        return pltpu.make_async_copy(
            table_hbm.at[idx_v.at[c]], rows_v.at[slot], sem_g.at[slot])

    def store_copy(c, slot):
        return pltpu.make_async_copy(
            rows_v.at[slot], out_hbm.at[pl.ds(base_row + c * CHUNK, CHUNK)],
            sem_s.at[slot])

    for c in range(LEAD):
        gather_copy(c, c % NBUF).start()

    def round_body(i, carry):
        for b in range(NBUF):
            k = i * NBUF + b
            kf = k + LEAD
            slot_f = (b + LEAD) % NBUF

            @pl.when(kf < NCHUNK)
            def _():
                @pl.when(kf >= NBUF)
                def _():
                    store_copy(kf - NBUF, slot_f).wait()
                gather_copy(kf, slot_f).start()

            gather_copy(k, b).wait()
            phase = lax.rem(k * CHUNK, SEQ)

            def add_pos(r, carry):
                for v in range(DIM // LANES):
                    vec = pos_v[phase + r, pl.ds(v * LANES, LANES)]
                    plsc.addupdate(rows_v.at[b, r, pl.ds(v * LANES, LANES)], vec)
                return carry

            lax.fori_loop(0, CHUNK, add_pos, 0, unroll=4)
            store_copy(k, b).start()
        return carry

    lax.fori_loop(0, NCHUNK // NBUF, round_body, 0)

    for b in range(NBUF):
        store_copy(NCHUNK - NBUF + b, b).wait()


def kernel(inputs, word_table, pos_table):
    idx = inputs.astype(jnp.int32).reshape(NW * NCHUNK, CHUNK)
    out = _emb_kernel(idx, pos_table, word_table)
    return out.reshape(BATCH, SEQ, DIM)


# speedup vs baseline: 1.2183x; 1.2183x over previous
import functools

import jax
import jax.numpy as jnp
from jax import lax
from jax.experimental import pallas as pl
from jax.experimental.pallas import tpu as pltpu
from jax.experimental.pallas import tpu_sc as plsc

VOCAB = 1000000
SEQ = 200
DIM = 64
BATCH = 4096

NC = 2
NS = 16
NW = NC * NS
LANES = 16

BPW = BATCH // NW
SPLIT = 104
NBUF = 4
LEAD = 2


@functools.partial(
    pl.kernel,
    mesh=plsc.VectorSubcoreMesh(core_axis_name="c", subcore_axis_name="s"),
    out_type=jax.ShapeDtypeStruct((BATCH, SEQ, DIM), jnp.float32),
    compiler_params=pltpu.CompilerParams(use_tc_tiling_on_sc=False),
    scratch_types=[
        pltpu.VMEM((BPW, SEQ), jnp.int32),
        pltpu.VMEM((SEQ, DIM), jnp.float32),
        pltpu.VMEM((NBUF, SEQ, DIM), jnp.float32),
        pltpu.SemaphoreType.DMA((NBUF,)),
        pltpu.SemaphoreType.DMA((NBUF,)),
    ],
)
def _emb_kernel(idx_hbm, pos_hbm, table_hbm, out_hbm,
                idx_v, pos_v, rows_v, sem_g, sem_s):
    wid = lax.axis_index("s") * NC + lax.axis_index("c")
    row0 = wid * BPW

    pltpu.sync_copy(idx_hbm.at[pl.ds(row0, BPW)], idx_v)
    pltpu.sync_copy(pos_hbm, pos_v)

    def gather_copies(r, slot):
        return (
            pltpu.make_async_copy(
                table_hbm.at[idx_v.at[r, pl.ds(0, SPLIT)]],
                rows_v.at[slot, pl.ds(0, SPLIT)], sem_g.at[slot]),
            pltpu.make_async_copy(
                table_hbm.at[idx_v.at[r, pl.ds(SPLIT, SEQ - SPLIT)]],
                rows_v.at[slot, pl.ds(SPLIT, SEQ - SPLIT)], sem_g.at[slot]),
        )

    def store_copy(r, slot):
        return pltpu.make_async_copy(
            rows_v.at[slot], out_hbm.at[row0 + r], sem_s.at[slot])

    for r in range(LEAD):
        for c in gather_copies(r, r % NBUF):
            c.start()

    def row_body(i, carry):
        for b in range(NBUF):
            r = i * NBUF + b
            rf = r + LEAD
            slot_f = (b + LEAD) % NBUF

            @pl.when(rf < BPW)
            def _():
                @pl.when(rf >= NBUF)
                def _():
                    store_copy(rf - NBUF, slot_f).wait()
                for c in gather_copies(rf, slot_f):
                    c.start()

            for c in gather_copies(r, b):
                c.wait()

            def add_pos(j, inner):
                for v in range(DIM // LANES):
                    vec = pos_v[j, pl.ds(v * LANES, LANES)]
                    plsc.addupdate(rows_v.at[b, j, pl.ds(v * LANES, LANES)], vec)
                return inner

            lax.fori_loop(0, SEQ, add_pos, 0, unroll=4)
            store_copy(r, b).start()
        return carry

    lax.fori_loop(0, BPW // NBUF, row_body, 0)

    for b in range(NBUF):
        store_copy(BPW - NBUF + b, b).wait()


def kernel(inputs, word_table, pos_table):
    return _emb_kernel(inputs.astype(jnp.int32), pos_table, word_table)
